# Initial kernel scaffold; baseline (speedup 1.0000x reference)
import jax
import jax.numpy as jnp
from jax.experimental import pallas as pl


def kernel(z, neighbor_indices, adjacency, basis_weights, channel_coeffs):
    raise NotImplementedError("write your pallas kernel here")


# speedup vs baseline: 1.9713x; 1.9713x over previous
import functools

import jax
import jax.numpy as jnp
from jax import lax
from jax.experimental import pallas as pl
from jax.experimental.pallas import tpu as pltpu
from jax.experimental.pallas import tpu_sc as plsc

L = 16


def _build_sc_kernel(N, NP, TC, K, M, C, T, G, per_w, NC, NS):
    chunks = per_w // G
    CV = C // L
    mesh = plsc.VectorSubcoreMesh(core_axis_name="c", subcore_axis_name="s")

    @functools.partial(
        pl.kernel,
        mesh=mesh,
        out_type=jax.ShapeDtypeStruct((NP, TC), jnp.float32),
        scratch_types=[
            pltpu.VMEM((G * K,), jnp.int32),
            pltpu.VMEM((G * K, TC), jnp.float32),
            pltpu.VMEM((G, 1 + M, K), jnp.float32),
            pltpu.VMEM((M, C), jnp.float32),
            pltpu.VMEM((G, TC), jnp.float32),
            pltpu.SemaphoreType.DMA,
        ],
    )
    def sck(z_hbm, idx_hbm, wgt_hbm, cc_hbm, out_hbm,
            idx_v, rows_v, wgt_v, cc_v, out_v, sem):
        wid = lax.axis_index("s") * NC + lax.axis_index("c")
        base = wid * per_w
        pltpu.sync_copy(cc_hbm, cc_v)

        def chunk_body(g, carry):
            n0 = base + g * G
            pltpu.sync_copy(idx_hbm.at[pl.ds(n0 * K, G * K)], idx_v)
            pltpu.async_copy(z_hbm.at[idx_v], rows_v, sem).wait()
            pltpu.sync_copy(wgt_hbm.at[pl.ds(n0, G)], wgt_v)

            def node_body(i, c2):
                adj_r = wgt_v[i, 0, :]
                a = [wgt_v[i, 1 + m, :] * adj_r for m in range(M)]
                ccl = [[cc_v[m, pl.ds(cv * L, L)] for cv in range(CV)]
                       for m in range(M)]
                acc = [[jnp.zeros((L,), jnp.float32) for _ in range(CV)]
                       for _ in range(T)]
                for j in range(K):
                    jf = jnp.full((L,), j, jnp.int32)
                    ab = [a[m].at[jf].get(mode="promise_in_bounds")
                          for m in range(M)]
                    for cv in range(CV):
                        w = ab[0] * ccl[0][cv]
                        for m in range(1, M):
                            w = w + ab[m] * ccl[m][cv]
                        for t in range(T):
                            zr = rows_v[i * K + j, pl.ds(t * C + cv * L, L)]
                            acc[t][cv] = acc[t][cv] + w * zr
                for t in range(T):
                    for cv in range(CV):
                        x = acc[t][cv]
                        e = jnp.exp(x + x)
                        out_v[i, pl.ds(t * C + cv * L, L)] = (
                            1.0 - 2.0 / (e + 1.0))
                return c2

            lax.fori_loop(0, G, node_body, 0)
            pltpu.sync_copy(out_v, out_hbm.at[pl.ds(n0, G)])
            return carry

        lax.fori_loop(0, chunks, chunk_body, 0)

    return sck


def kernel(z, neighbor_indices, adjacency, basis_weights, channel_coeffs):
    B, N, T, C = z.shape
    K = neighbor_indices.shape[1]
    M = basis_weights.shape[0]
    TC = T * C
    NC, NS = 2, 16
    NW = NC * NS
    G = 8
    per_w = -(-N // (NW * G)) * G
    NP = per_w * NW

    z_flat = z.reshape(B * N, TC)
    idx = neighbor_indices.astype(jnp.int32).reshape(-1)
    idx_p = jnp.pad(idx, (0, (NP - N) * K))
    adj = adjacency[:, :K]
    bas = jnp.transpose(basis_weights[:, :, :K], (1, 0, 2))
    wgt = jnp.concatenate([adj[:, None, :], bas], axis=1)
    wgt_p = jnp.pad(wgt, ((0, NP - N), (0, 0), (0, 0)))
    cc_t = channel_coeffs.T.astype(jnp.float32)

    sck = _build_sc_kernel(N, NP, TC, K, M, C, T, G, per_w, NC, NS)
    out = sck(z_flat, idx_p, wgt_p, cc_t)
    return out[:N].reshape(B, N, T, C)


# speedup vs baseline: 2.6559x; 1.3473x over previous
import functools

import jax
import jax.numpy as jnp
from jax import lax
from jax.experimental import pallas as pl
from jax.experimental.pallas import tpu as pltpu
from jax.experimental.pallas import tpu_sc as plsc

L = 16


def _build_sc_kernel(NP, TC, K, M, C, T, G, S, per_w, NC):
    chunks = per_w // G
    CV = C // L
    CVH = CV // 2
    R = G * K // S
    mesh = plsc.VectorSubcoreMesh(core_axis_name="c", subcore_axis_name="s")

    @functools.partial(
        pl.kernel,
        mesh=mesh,
        out_type=jax.ShapeDtypeStruct((NP, TC), jnp.float32),
        scratch_types=[
            pltpu.VMEM((2, S, 1, R), jnp.int32),
            pltpu.VMEM((2, G * K, TC), jnp.float32),
            pltpu.VMEM((2, G, 1 + M, K), jnp.float32),
            pltpu.VMEM((M, C), jnp.float32),
            pltpu.VMEM((2, G, TC), jnp.float32),
            pltpu.SemaphoreType.DMA,
            pltpu.SemaphoreType.DMA,
            pltpu.SemaphoreType.DMA,
            pltpu.SemaphoreType.DMA,
            pltpu.SemaphoreType.DMA,
            pltpu.SemaphoreType.DMA,
            pltpu.SemaphoreType.DMA,
            pltpu.SemaphoreType.DMA,
        ],
    )
    def sck(z_hbm, idx_hbm, wgt_hbm, cc_hbm, out_hbm,
            idx_v, rows_v, wgt_v, cc_v, out_v,
            si0, si1, sr0, sr1, sw0, sw1, so0, so1):
        si = [si0, si1]
        sr = [sr0, sr1]
        sw = [sw0, sw1]
        so = [so0, so1]
        wid = lax.axis_index("s") * NC + lax.axis_index("c")
        base = wid * per_w
        base_irow = base * K // R
        pltpu.sync_copy(cc_hbm, cc_v)

        def idx_copy(g, b):
            return pltpu.make_async_copy(
                idx_hbm.at[pl.ds(base_irow + g * S, S)], idx_v.at[b], si[b])

        def gather_sub(b, s):
            return pltpu.make_async_copy(
                z_hbm.at[idx_v.at[b, s, 0]],
                rows_v.at[b, pl.ds(s * R, R)], sr[b])

        def gather_start(b):
            for s in range(S):
                gather_sub(b, s).start()

        def gather_wait(b):
            for s in range(S):
                gather_sub(b, s).wait()

        def wgt_copy(g, b):
            return pltpu.make_async_copy(
                wgt_hbm.at[pl.ds(base + g * G, G)], wgt_v.at[b], sw[b])

        def out_copy(g, b):
            return pltpu.make_async_copy(
                out_v.at[b], out_hbm.at[pl.ds(base + g * G, G)], so[b])

        idx_copy(0, 0).start()
        idx_copy(0, 0).wait()
        gather_start(0)
        wgt_copy(0, 0).start()
        idx_copy(1, 1).start()

        def compute_chunk(g, b):
            def node_body(i, c2):
                adj_r = wgt_v[b, i, 0, :]
                a = [wgt_v[b, i, 1 + m, :] * adj_r for m in range(M)]
                for half in range(2):
                    ccl = [[cc_v[m, pl.ds((half * CVH + cv) * L, L)]
                            for cv in range(CVH)] for m in range(M)]
                    acc = [[jnp.zeros((L,), jnp.float32)
                            for _ in range(CVH)] for _ in range(T)]
                    for j in range(K):
                        jf = jnp.full((L,), j, jnp.int32)
                        ab = [a[m].at[jf].get(mode="promise_in_bounds")
                              for m in range(M)]
                        for cv in range(CVH):
                            w = ab[0] * ccl[0][cv]
                            for m in range(1, M):
                                w = w + ab[m] * ccl[m][cv]
                            for t in range(T):
                                off = t * C + (half * CVH + cv) * L
                                zr = rows_v[b, i * K + j, pl.ds(off, L)]
                                acc[t][cv] = acc[t][cv] + w * zr
                    for t in range(T):
                        for cv in range(CVH):
                            x = acc[t][cv]
                            e = jnp.exp(x + x)
                            off = t * C + (half * CVH + cv) * L
                            out_v[b, i, pl.ds(off, L)] = 1.0 - 2.0 / (e + 1.0)
                return c2

            lax.fori_loop(0, G, node_body, 0)

        def loop_body(g2, carry):
            for bb in range(2):
                g = g2 * 2 + bb
                nb = 1 - bb

                @pl.when(g + 1 < chunks)
                def _prefetch():
                    idx_copy(g + 1, nb).wait()
                    gather_start(nb)
                    wgt_copy(g + 1, nb).start()

                gather_wait(bb)
                wgt_copy(g, bb).wait()

                @pl.when(g + 2 < chunks)
                def _idx_next():
                    idx_copy(g + 2, bb).start()

                @pl.when(g >= 2)
                def _drain_out():
                    out_copy(g - 2, bb).wait()

                compute_chunk(g, bb)
                out_copy(g, bb).start()
            return carry

        lax.fori_loop(0, chunks // 2, loop_body, 0)
        out_copy(chunks - 2, 0).wait()
        out_copy(chunks - 1, 1).wait()

    return sck


def kernel(z, neighbor_indices, adjacency, basis_weights, channel_coeffs):
    B, N, T, C = z.shape
    K = neighbor_indices.shape[1]
    M = basis_weights.shape[0]
    TC = T * C
    NC, NS = 2, 16
    NW = NC * NS
    G = 8
    S = 8
    per_w = -(-N // (NW * 2 * G)) * 2 * G
    NP = per_w * NW
    R = G * K // S

    z_flat = z.reshape(B * N, TC)
    idx = neighbor_indices.astype(jnp.int32).reshape(-1)
    idx_p = jnp.pad(idx, (0, (NP - N) * K)).reshape(-1, 1, R)
    adj = adjacency[:, :K]
    bas = jnp.transpose(basis_weights[:, :, :K], (1, 0, 2))
    wgt = jnp.concatenate([adj[:, None, :], bas], axis=1)
    wgt_p = jnp.pad(wgt, ((0, NP - N), (0, 0), (0, 0)))
    cc_t = channel_coeffs.T.astype(jnp.float32)

    sck = _build_sc_kernel(NP, TC, K, M, C, T, G, S, per_w, NC)
    out = sck(z_flat, idx_p, wgt_p, cc_t)
    return out[:N].reshape(B, N, T, C)
